# baseline (speedup 1.0000x reference)
import functools

import jax
import jax.numpy as jnp
from jax import lax
from jax.experimental import pallas as pl
from jax.experimental.pallas import tpu as pltpu
from jax.experimental.pallas import tpu_sc as plsc

_K = 2000
_KP = 2048
_N = 20000
_NP = 32768
_ROWS = _NP // 128
_IOU_T = 0.5
_DW = 16


def _ce(arrs, j, fwd, r_io, l_io):
    if j >= 128:
        jr = j // 128
        bit0 = (r_io & jr) == 0
        shuf = lambda x: jnp.where(bit0, jnp.roll(x, -jr, axis=0),
                                   jnp.roll(x, jr, axis=0))
    else:
        bit0 = (l_io & j) == 0
        shuf = lambda x: jnp.where(bit0, jnp.roll(x, -j, axis=1),
                                   jnp.roll(x, j, axis=1))
    p = [shuf(a) for a in arrs]
    before = (arrs[0] > p[0]) | ((arrs[0] == p[0]) & (arrs[1] < p[1]))
    keep_self = before == (bit0 == fwd)
    return [jnp.where(keep_self, a, b) for a, b in zip(arrs, p)]


def _iotas(rows):
    r_io = jax.lax.broadcasted_iota(jnp.int32, (rows, 128), 0)
    l_io = jax.lax.broadcasted_iota(jnp.int32, (rows, 128), 1)
    return r_io, l_io, r_io * 128 + l_io


def _topk_body(sc_ref, sc_out, idx_out):
    r_io, l_io, flat = _iotas(_ROWS)
    arrs = [sc_ref[...], flat]

    kk = 2
    while kk <= _KP:
        fwd = (flat & kk) == 0
        j = kk // 2
        while j >= 1:
            arrs = _ce(arrs, j, fwd, r_io, l_io)
            j //= 2
        kk *= 2

    rows = _ROWS
    while rows > 16:
        nblk = rows // 16
        a = [jnp.concatenate([x[32 * m:32 * m + 16] for m in range(nblk // 2)],
                             axis=0) for x in arrs]
        b = [jnp.concatenate([x[32 * m + 16:32 * m + 32] for m in range(nblk // 2)],
                             axis=0) for x in arrs]
        win = (a[0] > b[0]) | ((a[0] == b[0]) & (a[1] < b[1]))
        arrs = [jnp.where(win, x, y) for x, y in zip(a, b)]
        rows //= 2
        r_io, l_io, flat = _iotas(rows)
        fwd = (flat & _KP) == 0
        j = _KP // 2
        while j >= 1:
            arrs = _ce(arrs, j, fwd, r_io, l_io)
            j //= 2

    sc_out[...] = arrs[0]
    idx_out[...] = arrs[1]


_sc_info = plsc.get_sparse_core_info()
_NW = _sc_info.num_cores * _sc_info.num_subcores
_BPW = _KP // _NW


@functools.partial(
    pl.kernel,
    mesh=plsc.VectorSubcoreMesh(core_axis_name="c", subcore_axis_name="s"),
    out_type=jax.ShapeDtypeStruct((_KP, _DW), jnp.float32),
    compiler_params=pltpu.CompilerParams(use_tc_tiling_on_sc=False),
    scratch_types=[
        pltpu.VMEM((_BPW,), jnp.int32),
        pltpu.VMEM((_BPW, _DW), jnp.float32),
        pltpu.SemaphoreType.DMA,
    ],
)
def _gather_rows(table_hbm, idx_hbm, out_hbm, idx_v, rows_v, sem):
    wid = lax.axis_index("s") * _sc_info.num_cores + lax.axis_index("c")
    base = wid * _BPW
    pltpu.sync_copy(idx_hbm.at[pl.ds(base, _BPW)], idx_v)
    pltpu.async_copy(table_hbm.at[idx_v], rows_v, sem).wait()
    pltpu.sync_copy(rows_v, out_hbm.at[pl.ds(base, _BPW)])


def _nms_body(b_col_ref, b_row_ref, sc_ref, out_ref):
    x1j = b_col_ref[:, 0:1]
    y1j = b_col_ref[:, 1:2]
    x2j = b_col_ref[:, 2:3]
    y2j = b_col_ref[:, 3:4]
    x1i = b_row_ref[0:1, :]
    y1i = b_row_ref[1:2, :]
    x2i = b_row_ref[2:3, :]
    y2i = b_row_ref[3:4, :]

    xx1 = jnp.maximum(x1j, x1i)
    yy1 = jnp.maximum(y1j, y1i)
    xx2 = jnp.minimum(x2j, x2i)
    yy2 = jnp.minimum(y2j, y2i)
    inter = jnp.clip(xx2 - xx1, 0.0) * jnp.clip(yy2 - yy1, 0.0)
    area_j = jnp.clip(x2j - x1j, 0.0) * jnp.clip(y2j - y1j, 0.0)
    area_i = jnp.clip(x2i - x1i, 0.0) * jnp.clip(y2i - y1i, 0.0)
    union = area_j + area_i - inter
    iou = inter / jnp.maximum(union, 1e-8)

    jdx = jax.lax.broadcasted_iota(jnp.int32, (_KP, _KP), 0)
    idx = jax.lax.broadcasted_iota(jnp.int32, (_KP, _KP), 1)
    s_mat = jnp.where((iou > _IOU_T) & (jdx < idx), 1.0, 0.0)

    def cond(carry):
        return carry[1]

    def body(carry):
        k, _ = carry
        sup = jnp.dot(k, s_mat, preferred_element_type=jnp.float32)
        k_new = jnp.where(sup > 0.5, 0.0, 1.0)
        return k_new, jnp.any(k_new != k)

    k0 = jnp.ones((8, _KP), jnp.float32)
    k, _ = jax.lax.while_loop(cond, body, (k0, jnp.bool_(True)))
    krow = k[0:1, :]
    out_ref[0:4, :] = b_row_ref[...] * krow
    out_ref[4:5, :] = sc_ref[...] * krow


def kernel(boxes, scores):
    sc_plane = jnp.pad(scores, (0, _NP - _N), constant_values=-1.0)
    sc_plane = sc_plane.reshape(_ROWS, 128)

    sc16, idx16 = pl.pallas_call(
        _topk_body,
        out_shape=(jax.ShapeDtypeStruct((16, 128), jnp.float32),
                   jax.ShapeDtypeStruct((16, 128), jnp.int32)),
    )(sc_plane)

    table = jnp.pad(boxes, ((0, 0), (0, _DW - 4)))
    rows = _gather_rows(table, idx16.reshape(_KP))

    sc = sc16.reshape(1, _KP)
    b_col = rows[:, :4]
    b_row = b_col.T
    out5 = pl.pallas_call(
        _nms_body,
        out_shape=jax.ShapeDtypeStruct((5, _KP), jnp.float32),
    )(b_col, b_row, sc)
    return out5[:, :_K].T


# speedup vs baseline: 1.4690x; 1.4690x over previous
import jax
import jax.numpy as jnp
from jax.experimental import pallas as pl

_K = 2000
_KP = 2048
_N = 20000
_NP = 32768
_ROWS = _NP // 128
_IOU_T = 0.5


def _ce(arrs, j, fwd, r_io, l_io):
    if j >= 128:
        jr = j // 128
        bit0 = (r_io & jr) == 0
        shuf = lambda x: jnp.where(bit0, jnp.roll(x, -jr, axis=0),
                                   jnp.roll(x, jr, axis=0))
    else:
        bit0 = (l_io & j) == 0
        shuf = lambda x: jnp.where(bit0, jnp.roll(x, -j, axis=1),
                                   jnp.roll(x, j, axis=1))
    p = [shuf(a) for a in arrs]
    before = (arrs[0] > p[0]) | ((arrs[0] == p[0]) & (arrs[1] < p[1]))
    keep_self = before == (bit0 == fwd)
    return [jnp.where(keep_self, a, b) for a, b in zip(arrs, p)]


def _iotas(rows):
    r_io = jax.lax.broadcasted_iota(jnp.int32, (rows, 128), 0)
    l_io = jax.lax.broadcasted_iota(jnp.int32, (rows, 128), 1)
    return r_io, l_io, r_io * 128 + l_io


def _topk_body(sc_ref, bx_ref, out_ref):
    r_io, l_io, flat = _iotas(_ROWS)
    arrs = [sc_ref[...], flat, bx_ref[0], bx_ref[1], bx_ref[2], bx_ref[3]]

    kk = 2
    while kk <= _KP:
        fwd = (flat & kk) == 0
        j = kk // 2
        while j >= 1:
            arrs = _ce(arrs, j, fwd, r_io, l_io)
            j //= 2
        kk *= 2

    rows = _ROWS
    while rows > 16:
        nblk = rows // 16
        a = [jnp.concatenate([x[32 * m:32 * m + 16] for m in range(nblk // 2)],
                             axis=0) for x in arrs]
        b = [jnp.concatenate([x[32 * m + 16:32 * m + 32] for m in range(nblk // 2)],
                             axis=0) for x in arrs]
        win = (a[0] > b[0]) | ((a[0] == b[0]) & (a[1] < b[1]))
        arrs = [jnp.where(win, x, y) for x, y in zip(a, b)]
        rows //= 2
        r_io, l_io, flat = _iotas(rows)
        fwd = (flat & _KP) == 0
        j = _KP // 2
        while j >= 1:
            arrs = _ce(arrs, j, fwd, r_io, l_io)
            j //= 2

    out_ref[0] = arrs[0]
    out_ref[1] = arrs[2]
    out_ref[2] = arrs[3]
    out_ref[3] = arrs[4]
    out_ref[4] = arrs[5]


def _nms_body(b_col_ref, b_row_ref, sc_ref, out_ref):
    x1j = b_col_ref[:, 0:1]
    y1j = b_col_ref[:, 1:2]
    x2j = b_col_ref[:, 2:3]
    y2j = b_col_ref[:, 3:4]
    x1i = b_row_ref[0:1, :]
    y1i = b_row_ref[1:2, :]
    x2i = b_row_ref[2:3, :]
    y2i = b_row_ref[3:4, :]

    xx1 = jnp.maximum(x1j, x1i)
    yy1 = jnp.maximum(y1j, y1i)
    xx2 = jnp.minimum(x2j, x2i)
    yy2 = jnp.minimum(y2j, y2i)
    inter = jnp.clip(xx2 - xx1, 0.0) * jnp.clip(yy2 - yy1, 0.0)
    area_j = jnp.clip(x2j - x1j, 0.0) * jnp.clip(y2j - y1j, 0.0)
    area_i = jnp.clip(x2i - x1i, 0.0) * jnp.clip(y2i - y1i, 0.0)
    union = area_j + area_i - inter
    iou = inter / jnp.maximum(union, 1e-8)

    jdx = jax.lax.broadcasted_iota(jnp.int32, (_KP, _KP), 0)
    idx = jax.lax.broadcasted_iota(jnp.int32, (_KP, _KP), 1)
    s_mat = jnp.where((iou > _IOU_T) & (jdx < idx), 1.0, 0.0)

    def cond(carry):
        return carry[1]

    def body(carry):
        k, _ = carry
        sup = jnp.dot(k, s_mat, preferred_element_type=jnp.float32)
        k_new = jnp.where(sup > 0.5, 0.0, 1.0)
        return k_new, jnp.any(k_new != k)

    k0 = jnp.ones((8, _KP), jnp.float32)
    k, _ = jax.lax.while_loop(cond, body, (k0, jnp.bool_(True)))
    krow = k[0:1, :]
    out_ref[0:4, :] = b_row_ref[...] * krow
    out_ref[4:5, :] = sc_ref[...] * krow


def kernel(boxes, scores):
    sc_plane = jnp.pad(scores, (0, _NP - _N), constant_values=-1.0)
    sc_plane = sc_plane.reshape(_ROWS, 128)
    bx_planes = jnp.pad(boxes.T, ((0, 0), (0, _NP - _N)))
    bx_planes = bx_planes.reshape(4, _ROWS, 128)

    top5 = pl.pallas_call(
        _topk_body,
        out_shape=jax.ShapeDtypeStruct((5, 16, 128), jnp.float32),
    )(sc_plane, bx_planes)

    sc = top5[0].reshape(1, _KP)
    b_row = top5[1:5].reshape(4, _KP)
    b_col = b_row.T
    out5 = pl.pallas_call(
        _nms_body,
        out_shape=jax.ShapeDtypeStruct((5, _KP), jnp.float32),
    )(b_col, b_row, sc)
    return out5[:, :_K].T
